# baseline (speedup 1.0000x reference)
import functools

import jax
import jax.numpy as jnp
from jax import lax
from jax.experimental import pallas as pl
from jax.experimental.pallas import tpu as pltpu
from jax.experimental.pallas import tpu_sc as plsc

_VOCAB = 1000000
_HIDDEN = 64
_B = 4096
_L = 200
_N = _B * _L
_NW = 32
_PER_W = _N // _NW
_G = 128
_NG = _PER_W // _G
_K = 4
_NH = _NG // _K

_NBLK = _VOCAB // _G
_TAIL = _VOCAB - _NBLK * _G
_ITER1 = (_NBLK + _NW) // _NW


def _transpose_block(tb, ob, iota, j0_lo, nrow):
    for j0 in range(j0_lo, nrow, 16):
        base = (iota + j0) * _HIDDEN
        for a in range(8):
            for b in range(8):
                x = tb[a, b, pl.ds(j0, 16)]
                plsc.store_scatter(ob, [base + (8 * a + b)], x)


def _prep_body(tab3, tail3, scratch, tb_v, ob_v, isem, osem):
    wid = lax.axis_index("s") * 2 + lax.axis_index("c")
    iota = lax.iota(jnp.int32, 16)

    def step(i, carry):
        c = wid + i * _NW

        @pl.when(c < _NBLK)
        def _full():
            pltpu.async_copy(
                tab3.at[:, :, pl.ds(pl.multiple_of(c * _G, _G), _G)],
                tb_v,
                isem,
            ).wait()
            _transpose_block(tb_v, ob_v, iota, 0, _G)
            pltpu.async_copy(
                ob_v,
                scratch.at[
                    pl.ds(pl.multiple_of(c * _G * _HIDDEN, _G * _HIDDEN), _G * _HIDDEN)
                ],
                osem,
            ).wait()

        @pl.when(c == _NBLK)
        def _tail():
            pltpu.async_copy(tail3, tb_v, isem).wait()
            _transpose_block(tb_v, ob_v, iota, _G - _TAIL, _G)
            pltpu.async_copy(
                ob_v.at[pl.ds((_G - _TAIL) * _HIDDEN, _TAIL * _HIDDEN)],
                scratch.at[pl.ds(_NBLK * _G * _HIDDEN, _TAIL * _HIDDEN)],
                osem,
            ).wait()

        return carry

    lax.fori_loop(0, _ITER1, step, 0)


def _gather_body(idx_hbm, table_hbm, out_hbm, idx_v, rows_v, gsem, wsem):
    wid = lax.axis_index("s") * 2 + lax.axis_index("c")
    pltpu.sync_copy(idx_hbm.at[pl.ds(wid * _NG, _NG)], idx_v)
    grp_base = wid * _NG

    def outer(j2, carry):
        for jj in range(2):
            j = j2 * 2 + jj
            b0 = jj * _K
            half = rows_v.at[pl.ds(b0, _K)]

            @pl.when(j2 > 0)
            def _drain_prev_write():
                pltpu.make_async_copy(
                    half,
                    out_hbm.at[pl.ds(grp_base, _K), :, pl.ds(0, _HIDDEN)],
                    wsem,
                ).wait()

            for b in range(_K):
                g = j * _K + b
                pltpu.async_copy(
                    table_hbm.at[idx_v.at[g]], rows_v.at[b0 + b], gsem
                )
            for b in range(_K):
                pltpu.make_async_copy(
                    table_hbm.at[idx_v.at[0]], rows_v.at[b0], gsem
                ).wait()
            pltpu.async_copy(
                half,
                out_hbm.at[pl.ds(grp_base + j * _K, _K), :, pl.ds(0, _HIDDEN)],
                wsem,
            )
        return carry

    lax.fori_loop(0, _NH // 2, outer, 0)
    for jj in range(2):
        pltpu.make_async_copy(
            rows_v.at[pl.ds(jj * _K, _K)],
            out_hbm.at[pl.ds(grp_base, _K), :, pl.ds(0, _HIDDEN)],
            wsem,
        ).wait()


@jax.jit
def _run(ids, table):
    mesh = plsc.VectorSubcoreMesh(core_axis_name="c", subcore_axis_name="s")

    tab3 = table.T.reshape(8, 8, _VOCAB)
    tail3 = table.T[:, _VOCAB - _G:].reshape(8, 8, _G)
    prep = functools.partial(
        pl.kernel,
        mesh=mesh,
        out_type=jax.ShapeDtypeStruct((_VOCAB * _HIDDEN,), jnp.float32),
        scratch_types=[
            pltpu.VMEM((8, 8, _G), jnp.float32),
            pltpu.VMEM((_G * _HIDDEN,), jnp.float32),
            pltpu.SemaphoreType.DMA,
            pltpu.SemaphoreType.DMA,
        ],
        compiler_params=pltpu.CompilerParams(needs_layout_passes=False),
    )(_prep_body)
    scratch = prep(tab3, tail3)

    idx = ids.reshape(-1).astype(jnp.int32).reshape(_N // _G, _G)
    table_rm = scratch.reshape(_VOCAB, _HIDDEN)
    gather = functools.partial(
        pl.kernel,
        mesh=mesh,
        out_type=jax.ShapeDtypeStruct((_N // _G, _G, 2 * _HIDDEN), jnp.float32),
        scratch_types=[
            pltpu.VMEM((_NG, _G), jnp.int32),
            pltpu.VMEM((2 * _K, _G, _HIDDEN), jnp.float32),
            pltpu.SemaphoreType.DMA,
            pltpu.SemaphoreType.DMA,
        ],
        compiler_params=pltpu.CompilerParams(use_tc_tiling_on_sc=False),
    )(_gather_body)
    return gather(idx, table_rm)


def kernel(input_ids, attention_mask, embedding_weight):
    del attention_mask
    out = _run(input_ids, embedding_weight)
    return out[:, :, :_HIDDEN].reshape(_B, _L, _HIDDEN)


# speedup vs baseline: 1.2305x; 1.2305x over previous
import functools

import jax
import jax.numpy as jnp
from jax import lax
from jax.experimental import pallas as pl
from jax.experimental.pallas import tpu as pltpu
from jax.experimental.pallas import tpu_sc as plsc

_VOCAB = 1000000
_HIDDEN = 64
_B = 4096
_L = 200
_N = _B * _L
_NW = 32
_PER_W = _N // _NW
_G = 128
_NG = _PER_W // _G
_K = 4
_NH = _NG // _K

_NBLK = _VOCAB // _G
_TAIL = _VOCAB - _NBLK * _G
_ITER1 = (_NBLK + _NW) // _NW


def _transpose_buf(tb, ob, jbase, j0_range):
    def a_body(a, carry):
        f0 = a * 8
        for b in range(8):
            for j0 in j0_range:
                x = tb[a, b, pl.ds(16 * j0, 16)]
                plsc.store_scatter(ob, [jbase[j0] + (f0 + b)], x)
        return carry

    lax.fori_loop(0, 8, a_body, 0)


def _prep_body(tab3, tail3, scratch, tb0, tb1, ob0, ob1, isem, osem0, osem1):
    wid = lax.axis_index("s") * 2 + lax.axis_index("c")
    iota = lax.iota(jnp.int32, 16)
    jbase = [(iota + 16 * j0) * _HIDDEN for j0 in range(8)]

    def blk_src(c):
        return tab3.at[:, :, pl.ds(pl.multiple_of(c * _G, _G), _G)]

    tbs, obs = (tb0, tb1), (ob0, ob1)
    pltpu.async_copy(blk_src(wid), tb0, isem)

    def outer(i2, carry):
        for p in range(2):
            i = i2 * 2 + p
            c = wid + i * _NW

            @pl.when(c < _NBLK)
            def _do():
                pltpu.make_async_copy(blk_src(0), tbs[p], isem).wait()
                c2 = c + _NW

                @pl.when(c2 < _NBLK)
                def _prefetch():
                    pltpu.async_copy(blk_src(c2), tbs[1 - p], isem)

                osem = osem0 if p == 0 else osem1

                @pl.when(i2 > 0)
                def _drain_w():
                    pltpu.make_async_copy(
                        obs[p], scratch.at[pl.ds(0, _G * _HIDDEN)], osem
                    ).wait()

                _transpose_buf(tbs[p], obs[p], jbase, range(8))
                pltpu.async_copy(
                    obs[p],
                    scratch.at[
                        pl.ds(
                            pl.multiple_of(c * _G * _HIDDEN, _G * _HIDDEN),
                            _G * _HIDDEN,
                        )
                    ],
                    osem,
                )

        return carry

    lax.fori_loop(0, (_ITER1 + 1) // 2, outer, 0)
    for ob, osem in ((ob0, osem0), (ob1, osem1)):
        pltpu.make_async_copy(
            ob, scratch.at[pl.ds(0, _G * _HIDDEN)], osem
        ).wait()

    @pl.when(wid == 4)
    def _tail():
        pltpu.async_copy(tail3, tb0, isem).wait()
        _transpose_buf(tb0, ob0, jbase, range(4, 8))
        pltpu.async_copy(
            ob0.at[pl.ds((_G - _TAIL) * _HIDDEN, _TAIL * _HIDDEN)],
            scratch.at[pl.ds(_NBLK * _G * _HIDDEN, _TAIL * _HIDDEN)],
            osem0,
        ).wait()


def _gather_body(idx_hbm, table_hbm, out_hbm, idx_v, rows_v, gsem, wsem):
    wid = lax.axis_index("s") * 2 + lax.axis_index("c")
    pltpu.sync_copy(idx_hbm.at[pl.ds(wid * _NG, _NG)], idx_v)
    grp_base = wid * _NG

    def outer(j2, carry):
        for jj in range(2):
            j = j2 * 2 + jj
            b0 = jj * _K
            half = rows_v.at[pl.ds(b0, _K)]

            @pl.when(j2 > 0)
            def _drain_prev_write():
                pltpu.make_async_copy(
                    half,
                    out_hbm.at[pl.ds(grp_base, _K), :, pl.ds(0, _HIDDEN)],
                    wsem,
                ).wait()

            for b in range(_K):
                g = j * _K + b
                pltpu.async_copy(
                    table_hbm.at[idx_v.at[g]], rows_v.at[b0 + b], gsem
                )
            for b in range(_K):
                pltpu.make_async_copy(
                    table_hbm.at[idx_v.at[0]], rows_v.at[b0], gsem
                ).wait()
            pltpu.async_copy(
                half,
                out_hbm.at[pl.ds(grp_base + j * _K, _K), :, pl.ds(0, _HIDDEN)],
                wsem,
            )
        return carry

    lax.fori_loop(0, _NH // 2, outer, 0)
    for jj in range(2):
        pltpu.make_async_copy(
            rows_v.at[pl.ds(jj * _K, _K)],
            out_hbm.at[pl.ds(grp_base, _K), :, pl.ds(0, _HIDDEN)],
            wsem,
        ).wait()


@jax.jit
def _run(ids, table):
    mesh = plsc.VectorSubcoreMesh(core_axis_name="c", subcore_axis_name="s")

    tab3 = table.T.reshape(8, 8, _VOCAB)
    tail3 = table.T[:, _VOCAB - _G:].reshape(8, 8, _G)
    prep = functools.partial(
        pl.kernel,
        mesh=mesh,
        out_type=jax.ShapeDtypeStruct((_VOCAB * _HIDDEN,), jnp.float32),
        scratch_types=[
            pltpu.VMEM((8, 8, _G), jnp.float32),
            pltpu.VMEM((8, 8, _G), jnp.float32),
            pltpu.VMEM((_G * _HIDDEN,), jnp.float32),
            pltpu.VMEM((_G * _HIDDEN,), jnp.float32),
            pltpu.SemaphoreType.DMA,
            pltpu.SemaphoreType.DMA,
            pltpu.SemaphoreType.DMA,
        ],
        compiler_params=pltpu.CompilerParams(needs_layout_passes=False),
    )(_prep_body)
    scratch = prep(tab3, tail3)

    idx = ids.reshape(-1).astype(jnp.int32).reshape(_N // _G, _G)
    table_rm = scratch.reshape(_VOCAB, _HIDDEN)
    gather = functools.partial(
        pl.kernel,
        mesh=mesh,
        out_type=jax.ShapeDtypeStruct((_N // _G, _G, 2 * _HIDDEN), jnp.float32),
        scratch_types=[
            pltpu.VMEM((_NG, _G), jnp.int32),
            pltpu.VMEM((2 * _K, _G, _HIDDEN), jnp.float32),
            pltpu.SemaphoreType.DMA,
            pltpu.SemaphoreType.DMA,
        ],
        compiler_params=pltpu.CompilerParams(use_tc_tiling_on_sc=False),
    )(_gather_body)
    return gather(idx, table_rm)


def kernel(input_ids, attention_mask, embedding_weight):
    del attention_mask
    out = _run(input_ids, embedding_weight)
    return out[:, :, :_HIDDEN].reshape(_B, _L, _HIDDEN)


# speedup vs baseline: 1.7116x; 1.3910x over previous
import functools

import jax
import jax.numpy as jnp
from jax import lax
from jax.experimental import pallas as pl
from jax.experimental.pallas import tpu as pltpu
from jax.experimental.pallas import tpu_sc as plsc

_VOCAB = 1000000
_HIDDEN = 64
_B = 4096
_L = 200
_N = _B * _L
_NW = 32
_PER_W = _N // _NW
_G = 128
_NG = _PER_W // _G
_K = 4
_NH = _NG // _K

_NBLK = _VOCAB // _G
_TAIL = _VOCAB - _NBLK * _G
_ITER1 = (_NBLK + _NW) // _NW


def _tc_prep_body(in_ref, out_ref):
    x = in_ref[...]
    y = x.T
    out_ref[...] = jnp.concatenate([y, y], axis=1)


def _gather_body(idx_hbm, table_hbm, out_hbm, idx_v, rows_v, gsem, wsem):
    wid = lax.axis_index("s") * 2 + lax.axis_index("c")
    pltpu.sync_copy(idx_hbm.at[pl.ds(wid * _NG, _NG)], idx_v)
    grp_base = wid * _NG

    def outer(j2, carry):
        for jj in range(2):
            j = j2 * 2 + jj
            b0 = jj * _K
            half = rows_v.at[pl.ds(b0, _K)]

            @pl.when(j2 > 0)
            def _drain_prev_write():
                pltpu.make_async_copy(
                    half,
                    out_hbm.at[pl.ds(grp_base, _K), :, pl.ds(0, _HIDDEN)],
                    wsem,
                ).wait()

            for b in range(_K):
                g = j * _K + b
                pltpu.async_copy(
                    table_hbm.at[idx_v.at[g]], rows_v.at[b0 + b], gsem
                )
            for b in range(_K):
                pltpu.make_async_copy(
                    table_hbm.at[idx_v.at[0]], rows_v.at[b0], gsem
                ).wait()
            pltpu.async_copy(
                half,
                out_hbm.at[pl.ds(grp_base + j * _K, _K), :, pl.ds(0, _HIDDEN)],
                wsem,
            )
        return carry

    lax.fori_loop(0, _NH // 2, outer, 0)
    for jj in range(2):
        pltpu.make_async_copy(
            rows_v.at[pl.ds(jj * _K, _K)],
            out_hbm.at[pl.ds(grp_base, _K), :, pl.ds(0, _HIDDEN)],
            wsem,
        ).wait()


@jax.jit
def _run(ids, table):
    mesh = plsc.VectorSubcoreMesh(core_axis_name="c", subcore_axis_name="s")

    _CW = 1024
    tab_t = table.T
    prep = pl.pallas_call(
        _tc_prep_body,
        grid=((_VOCAB + _CW - 1) // _CW,),
        in_specs=[pl.BlockSpec((_HIDDEN, _CW), lambda c: (0, c))],
        out_specs=pl.BlockSpec((_CW, 2 * _HIDDEN), lambda c: (c, 0)),
        out_shape=jax.ShapeDtypeStruct((_VOCAB, 2 * _HIDDEN), jnp.float32),
    )
    scratch = prep(tab_t)

    idx = (ids.reshape(-1).astype(jnp.int32) * 2).reshape(_N // _G, _G)
    table_rm = scratch.reshape(2 * _VOCAB, _HIDDEN)
    gather = functools.partial(
        pl.kernel,
        mesh=mesh,
        out_type=jax.ShapeDtypeStruct((_N // _G, _G, 2 * _HIDDEN), jnp.float32),
        scratch_types=[
            pltpu.VMEM((_NG, _G), jnp.int32),
            pltpu.VMEM((2 * _K, _G, _HIDDEN), jnp.float32),
            pltpu.SemaphoreType.DMA,
            pltpu.SemaphoreType.DMA,
        ],
        compiler_params=pltpu.CompilerParams(use_tc_tiling_on_sc=False),
    )(_gather_body)
    return gather(idx, table_rm)


def kernel(input_ids, attention_mask, embedding_weight):
    del attention_mask
    out = _run(input_ids, embedding_weight)
    return out[:, :, :_HIDDEN].reshape(_B, _L, _HIDDEN)


# speedup vs baseline: 2.0434x; 1.1938x over previous
import functools

import jax
import jax.numpy as jnp
from jax import lax
from jax.experimental import pallas as pl
from jax.experimental.pallas import tpu as pltpu
from jax.experimental.pallas import tpu_sc as plsc

_VOCAB = 1000000
_HIDDEN = 64
_B = 4096
_L = 200
_N = _B * _L
_NW = 32
_PER_W = _N // _NW
_G = 128
_NG = _PER_W // _G
_K = 4
_NH = _NG // _K

_NBLK = _VOCAB // _G
_TAIL = _VOCAB - _NBLK * _G
_ITER1 = (_NBLK + _NW) // _NW


def _tc_prep_body(in_ref, out_ref):
    x = in_ref[...]
    y = x.T
    out_ref[...] = jnp.concatenate([y, y], axis=1)


def _gather_body(idx_hbm, table_hbm, out_hbm, idx_v, rows_v, gsem, wsem):
    wid = lax.axis_index("s") * 2 + lax.axis_index("c")
    pltpu.sync_copy(idx_hbm.at[pl.ds(wid * _NG, _NG)], idx_v)
    grp_base = wid * _NG

    def outer(j2, carry):
        for jj in range(2):
            j = j2 * 2 + jj
            b0 = jj * _K
            half = rows_v.at[pl.ds(b0, _K)]

            @pl.when(j2 > 0)
            def _drain_prev_write():
                pltpu.make_async_copy(
                    half,
                    out_hbm.at[pl.ds(grp_base, _K), :, pl.ds(0, _HIDDEN)],
                    wsem,
                ).wait()

            for b in range(_K):
                g = j * _K + b
                pltpu.async_copy(
                    table_hbm.at[idx_v.at[g]], rows_v.at[b0 + b], gsem
                )
            for b in range(_K):
                pltpu.make_async_copy(
                    table_hbm.at[idx_v.at[0]], rows_v.at[b0], gsem
                ).wait()
            pltpu.async_copy(
                half,
                out_hbm.at[pl.ds(grp_base + j * _K, _K), :, pl.ds(0, _HIDDEN)],
                wsem,
            )
        return carry

    lax.fori_loop(0, _NH // 2, outer, 0)
    for jj in range(2):
        pltpu.make_async_copy(
            rows_v.at[pl.ds(jj * _K, _K)],
            out_hbm.at[pl.ds(grp_base, _K), :, pl.ds(0, _HIDDEN)],
            wsem,
        ).wait()


@jax.jit
def _run(ids, table):
    mesh = plsc.VectorSubcoreMesh(core_axis_name="c", subcore_axis_name="s")

    r1 = lax.optimization_barrier(table.reshape(_VOCAB // 2, 2 * _HIDDEN))
    table_rm = r1.reshape(_VOCAB, _HIDDEN)

    idx = ids.reshape(-1).astype(jnp.int32).reshape(_N // _G, _G)
    gather = functools.partial(
        pl.kernel,
        mesh=mesh,
        out_type=jax.ShapeDtypeStruct((_N // _G, _G, 2 * _HIDDEN), jnp.float32),
        scratch_types=[
            pltpu.VMEM((_NG, _G), jnp.int32),
            pltpu.VMEM((2 * _K, _G, _HIDDEN), jnp.float32),
            pltpu.SemaphoreType.DMA,
            pltpu.SemaphoreType.DMA,
        ],
        compiler_params=pltpu.CompilerParams(use_tc_tiling_on_sc=False),
    )(_gather_body)
    return gather(idx, table_rm)


def kernel(input_ids, attention_mask, embedding_weight):
    del attention_mask
    out = _run(input_ids, embedding_weight)
    return out[:, :, :_HIDDEN].reshape(_B, _L, _HIDDEN)
